# baseline (speedup 1.0000x reference)
import functools
import math

import jax
import jax.numpy as jnp
from jax import lax
from jax.experimental import pallas as pl
from jax.experimental.pallas import tpu as pltpu
from jax.experimental.pallas import tpu_sc as plsc

_VOCAB = 1000
_SEQ = 200
_BATCH = 16384
_NW = 32
_RPT = _BATCH // _NW
_NCHUNK = 8
_RPC = _RPT // _NCHUNK
_CHUNK_W = _RPC * _SEQ
_TILE_W = _RPT * _SEQ
_NV = _SEQ // 16 + 1
_LOG_SEQ = math.log(float(_SEQ))
_LN2 = math.log(2.0)
_C5 = 0.043428907822058785
_C4 = -0.4048671744185487
_C3 = 1.5939013634971746
_C2 = -3.492494279876412
_C1 = 5.046876044973777
_C0 = -2.786812953866816


def _log_poly(cnt):
    f = cnt.astype(jnp.float32)
    b = plsc.bitcast(f, jnp.int32)
    e = ((b >> 23) - 127).astype(jnp.float32)
    m = plsc.bitcast((b & 0x7FFFFF) | 0x3F800000, jnp.float32)
    p = _C5
    for coef in (_C4, _C3, _C2, _C1, _C0):
        p = p * m + coef
    return (p + e) * _LN2


def _entropy_sc(x_hbm, out_hbm, tokens_v, hist_v, hist2_v, out_v, sems):
    wid = lax.axis_index("s") * 2 + lax.axis_index("c")
    base_w = wid * _TILE_W

    copies = []
    for ci in range(_NCHUNK):
        copies.append(
            pltpu.async_copy(
                x_hbm.at[pl.ds(base_w + ci * _CHUNK_W, _CHUNK_W)],
                tokens_v.at[pl.ds(ci * _CHUNK_W, _CHUNK_W)],
                sems.at[ci],
            )
        )
    lane = lax.iota(jnp.int32, 16)
    zeros_i = jnp.zeros((16,), jnp.int32)
    ones_i = jnp.ones((16,), jnp.int32)
    dummy = _VOCAB + lane
    tail_sel = lane < 8
    lane0 = lane == 0

    def zero_hist(k, carry):
        hist_v[pl.ds(k * 16, 16)] = zeros_i
        hist2_v[pl.ds(k * 16, 16)] = zeros_i
        return carry

    lax.fori_loop(0, _HIST_W // 16, zero_hist, 0)

    def load_row(r):
        rbase = r * _SEQ
        toks = [tokens_v[pl.ds(rbase + 16 * k, 16)] for k in range(_NV - 1)]
        toks.append(
            jnp.where(tail_sel, tokens_v[pl.ds(rbase + _SEQ - 8, 16)], dummy)
        )
        return toks

    def per_row_pair(p, carry):
        r = p * 2
        toks_a = load_row(r)
        toks_b = load_row(r + 1)

        for ta, tb in zip(toks_a, toks_b):
            plsc.addupdate_scatter(hist_v, [ta], ones_i)
            plsc.addupdate_scatter(hist2_v, [tb], ones_i)

        zf = jnp.zeros((16,), jnp.float32)
        accs_a = [zf, zf, zf, zf]
        accs_b = [zf, zf, zf, zf]
        for k, (ta, tb) in enumerate(zip(toks_a, toks_b)):
            cnt_a = plsc.load_gather(hist_v, [ta])
            cnt_b = plsc.load_gather(hist2_v, [tb])
            accs_a[k % 4] = accs_a[k % 4] + _log_poly(cnt_a)
            accs_b[k % 4] = accs_b[k % 4] + _log_poly(cnt_b)

        for ta, tb in zip(toks_a, toks_b):
            plsc.store_scatter(hist_v, [ta], zeros_i)
            plsc.store_scatter(hist2_v, [tb], zeros_i)

        s_a = jnp.sum((accs_a[0] + accs_a[1]) + (accs_a[2] + accs_a[3]))
        s_b = jnp.sum((accs_b[0] + accs_b[1]) + (accs_b[2] + accs_b[3]))
        h_a = _LOG_SEQ - s_a * (1.0 / _SEQ)
        h_b = _LOG_SEQ - s_b * (1.0 / _SEQ)
        plsc.store_scatter(out_v, [zeros_i + r],
                           jnp.zeros((16,), jnp.float32) + h_a, mask=lane0)
        plsc.store_scatter(out_v, [zeros_i + (r + 1)],
                           jnp.zeros((16,), jnp.float32) + h_b, mask=lane0)
        return carry

    for ci in range(_NCHUNK):
        copies[ci].wait()
        lax.fori_loop(
            0, _RPC // 2,
            lambda pp, c, ci=ci: per_row_pair(ci * (_RPC // 2) + pp, c), 0,
        )

    pltpu.sync_copy(out_v, out_hbm.at[pl.ds(wid * _RPT, _RPT)])


_HIST_W = 1024


def kernel(x):
    mesh = plsc.VectorSubcoreMesh(core_axis_name="c", subcore_axis_name="s")
    run = functools.partial(
        pl.kernel,
        mesh=mesh,
        out_type=jax.ShapeDtypeStruct((_BATCH,), jnp.float32),
        scratch_types=[
            pltpu.VMEM((_TILE_W + 16,), jnp.int32),
            pltpu.VMEM((_HIST_W,), jnp.int32),
            pltpu.VMEM((_HIST_W,), jnp.int32),
            pltpu.VMEM((_RPT,), jnp.float32),
            pltpu.SemaphoreType.DMA((_NCHUNK,)),
        ],
        compiler_params=pltpu.CompilerParams(needs_layout_passes=False),
    )(_entropy_sc)
    return run(x.reshape(-1))[:, None]
